# baseline (speedup 1.0000x reference)
import functools

import jax
import jax.numpy as jnp
from jax import lax
from jax.experimental import pallas as pl
from jax.experimental.pallas import tpu as pltpu
from jax.experimental.pallas import tpu_sc as plsc

B = 16384
D = 64
FU = 16
FI = 16
NROWS = 1000000
PROWS = NROWS // 2

_info = plsc.get_sparse_core_info()
_NC, _NS = _info.num_cores, _info.num_subcores
_NW = _NC * _NS
_BPW = B // _NW
_CHUNK = 128

_mesh = plsc.VectorSubcoreMesh(core_axis_name="c", subcore_axis_name="s")


_PC = 2048
_NPAIR = (NROWS + 2 * _PC - 1) // (2 * _PC)


def _project_body(tt_lo, tt_hi, we, out):
    dn = (((0,), (0,)), ((), ()))
    lo = lax.dot_general(tt_lo[...], we[...], dn,
                         preferred_element_type=jnp.float32)
    hi = lax.dot_general(tt_hi[...], we[...], dn,
                         preferred_element_type=jnp.float32)
    row = jnp.concatenate([lo, hi], axis=1)
    u = lax.bitcast_convert_type(row.astype(jnp.bfloat16),
                                 jnp.uint16).astype(jnp.uint32)
    u3 = u.reshape(_PC // 2, 2, 2 * D)
    packed = u3[:, 0, :] | (u3[:, 1, :] << 16)
    out[...] = lax.bitcast_convert_type(packed, jnp.float32)


def _project(table_t, we):
    return pl.pallas_call(
        _project_body,
        grid=(_NPAIR,),
        in_specs=[
            pl.BlockSpec((D, _PC), lambda i: (0, 2 * i)),
            pl.BlockSpec((D, _PC),
                         lambda i: (0, jnp.minimum(2 * i + 1, 2 * _NPAIR - 2))),
            pl.BlockSpec((D, D), lambda i: (0, 0)),
        ],
        out_specs=pl.BlockSpec((_PC // 2, 2 * D), lambda i: (i, 0)),
        out_shape=jax.ShapeDtypeStruct((_NPAIR * _PC // 2, 2 * D), jnp.float32),
    )(table_t, table_t, we)


@functools.partial(
    pl.kernel,
    mesh=_mesh,
    compiler_params=pltpu.CompilerParams(use_tc_tiling_on_sc=True),
    out_type=jax.ShapeDtypeStruct((B, 2 * D), jnp.float32),
    scratch_types=[
        pltpu.VMEM((_BPW,), jnp.int32),
        pltpu.VMEM((_BPW, 2 * D), jnp.float32),
        pltpu.SemaphoreType.DMA,
    ],
)
def _sc_gather(tbl_hbm, id_hbm, out_hbm, ids_v, rows_v, sem):
    wid = lax.axis_index("s") * _NC + lax.axis_index("c")
    base = wid * _BPW
    pltpu.sync_copy(id_hbm.at[pl.ds(base, _BPW)], ids_v)
    descs = []
    for k in range(0, _BPW, _CHUNK):
        descs.append(pltpu.async_copy(
            tbl_hbm.at[ids_v.at[pl.ds(k, _CHUNK)]],
            rows_v.at[pl.ds(k, _CHUNK)], sem))
    for d in descs:
        d.wait()
    pltpu.sync_copy(rows_v, out_hbm.at[pl.ds(base, _BPW)])


def _tail_body(up, uf, usub, upar, ip, if_, isub, ipar, wuf, bu, wif, bi,
               gu, betau, gi, betai, uo, io):
    def tower(packed, sub, par, feats, wf, b, g, beta, out):
        u = lax.bitcast_convert_type(packed[...], jnp.uint32)
        sel = jnp.where(sub[...] > 0, u >> 16, u & jnp.uint32(0xFFFF))
        row = lax.bitcast_convert_type(
            sel.astype(jnp.uint16), jnp.bfloat16).astype(jnp.float32)
        x = (jnp.where(par[...] > 0, row[:, D:], row[:, :D])
             + jnp.dot(feats[...], wf[...], preferred_element_type=jnp.float32)
             + b[...])
        mu = jnp.mean(x, axis=-1, keepdims=True)
        xc = x - mu
        var = jnp.mean(xc * xc, axis=-1, keepdims=True)
        out[...] = xc * lax.rsqrt(var + 1e-5) * g[...] + beta[...]

    tower(up, usub[...], upar[...], uf, wuf, bu, gu, betau, uo)
    tower(ip, isub[...], ipar[...], if_, wif, bi, gi, betai, io)


_BLK = 2048


def _tail(u_pairs, u_sub, u_par, u_feats, i_pairs, i_sub, i_par, i_feats,
          Wuf, bu, Wif, bi, gu, beta_u, gi, beta_i):
    grid = (B // _BLK,)
    pair_spec = pl.BlockSpec((_BLK, 2 * D), lambda i: (i, 0))
    par_spec = pl.BlockSpec((_BLK, 1), lambda i: (i, 0))
    row_spec = pl.BlockSpec((_BLK, D), lambda i: (i, 0))
    ufeat_spec = pl.BlockSpec((_BLK, FU), lambda i: (i, 0))
    ifeat_spec = pl.BlockSpec((_BLK, FI), lambda i: (i, 0))
    full = lambda shape: pl.BlockSpec(shape, lambda i: (0, 0))
    return pl.pallas_call(
        _tail_body,
        grid=grid,
        in_specs=[
            pair_spec, ufeat_spec, par_spec, par_spec,
            pair_spec, ifeat_spec, par_spec, par_spec,
            full((FU, D)), full((1, D)),
            full((FI, D)), full((1, D)),
            full((1, D)), full((1, D)), full((1, D)), full((1, D)),
        ],
        out_specs=[row_spec, row_spec],
        out_shape=[
            jax.ShapeDtypeStruct((B, D), jnp.float32),
            jax.ShapeDtypeStruct((B, D), jnp.float32),
        ],
    )(u_pairs, u_feats, u_sub, u_par, i_pairs, i_feats, i_sub, i_par,
      Wuf, bu, Wif, bi, gu, beta_u, gi, beta_i)


def kernel(user_ids, item_ids, user_feats, item_feats, user_table, item_table,
           Wu, bu, Wi, bi, gu, beta_u, gi, beta_i):
    uid = user_ids.astype(jnp.int32)
    iid = item_ids.astype(jnp.int32)
    pu = _project(user_table.T, Wu[:D])
    pi = _project(item_table.T, Wi[:D])
    u_row = ((uid >> 12) << 10) | ((uid & (_PC - 1)) >> 1)
    i_row = ((iid >> 12) << 10) | ((iid & (_PC - 1)) >> 1)
    u_pairs = _sc_gather(pu, u_row)
    i_pairs = _sc_gather(pi, i_row)
    u_sub = (uid & 1).reshape(B, 1)
    i_sub = (iid & 1).reshape(B, 1)
    u_par = ((uid >> 11) & 1).reshape(B, 1)
    i_par = ((iid >> 11) & 1).reshape(B, 1)
    return _tail(
        u_pairs, u_sub, u_par, user_feats,
        i_pairs, i_sub, i_par, item_feats,
        Wu[D:], bu.reshape(1, D),
        Wi[D:], bi.reshape(1, D),
        gu.reshape(1, D), beta_u.reshape(1, D),
        gi.reshape(1, D), beta_i.reshape(1, D),
    )


# speedup vs baseline: 1.6224x; 1.6224x over previous
import functools

import jax
import jax.numpy as jnp
from jax import lax
from jax.experimental import pallas as pl
from jax.experimental.pallas import tpu as pltpu
from jax.experimental.pallas import tpu_sc as plsc

B = 16384
D = 64
FU = 16
FI = 16
NROWS = 1000000
PROWS = NROWS // 2

_info = plsc.get_sparse_core_info()
_NC, _NS = _info.num_cores, _info.num_subcores
_NW = _NC * _NS
_BPW = B // _NW
_CHUNK = 128

_mesh = plsc.VectorSubcoreMesh(core_axis_name="c", subcore_axis_name="s")


_PC = 2048
_NG = (NROWS + 4 * _PC - 1) // (4 * _PC)
_LAST = (NROWS + _PC - 1) // _PC - 1


def _project_body(t0, t1, t2, t3, we, out):
    dn = (((0,), (0,)), ((), ()))

    def mm(t):
        return lax.dot_general(t[...], we[...], dn,
                               preferred_element_type=jnp.float32)

    def b16(x):
        return lax.bitcast_convert_type(
            x.astype(jnp.bfloat16), jnp.uint16).astype(jnp.uint32)

    x = b16(jnp.concatenate([mm(t0), mm(t1)], axis=1))
    y = b16(jnp.concatenate([mm(t2), mm(t3)], axis=1))
    out[...] = lax.bitcast_convert_type(x | (y << 16), jnp.float32)


def _project(table_t, we):
    def spec(k):
        return pl.BlockSpec(
            (D, _PC), lambda i: (0, jnp.minimum(4 * i + k, _LAST)))

    return pl.pallas_call(
        _project_body,
        grid=(_NG,),
        in_specs=[spec(0), spec(1), spec(2), spec(3),
                  pl.BlockSpec((D, D), lambda i: (0, 0))],
        out_specs=pl.BlockSpec((_PC, 2 * D), lambda i: (i, 0)),
        out_shape=jax.ShapeDtypeStruct((_NG * _PC, 2 * D), jnp.float32),
    )(table_t, table_t, table_t, table_t, we)


@functools.partial(
    pl.kernel,
    mesh=_mesh,
    compiler_params=pltpu.CompilerParams(use_tc_tiling_on_sc=True),
    out_type=jax.ShapeDtypeStruct((B, 2 * D), jnp.float32),
    scratch_types=[
        pltpu.VMEM((_BPW,), jnp.int32),
        pltpu.VMEM((_BPW, 2 * D), jnp.float32),
        pltpu.SemaphoreType.DMA,
    ],
)
def _sc_gather(tbl_hbm, id_hbm, out_hbm, ids_v, rows_v, sem):
    wid = lax.axis_index("s") * _NC + lax.axis_index("c")
    base = wid * _BPW
    pltpu.sync_copy(id_hbm.at[pl.ds(base, _BPW)], ids_v)
    descs = []
    for k in range(0, _BPW, _CHUNK):
        descs.append(pltpu.async_copy(
            tbl_hbm.at[ids_v.at[pl.ds(k, _CHUNK)]],
            rows_v.at[pl.ds(k, _CHUNK)], sem))
    for d in descs:
        d.wait()
    pltpu.sync_copy(rows_v, out_hbm.at[pl.ds(base, _BPW)])


def _tail_body(up, uf, usub, upar, ip, if_, isub, ipar, wuf, bu, wif, bi,
               gu, betau, gi, betai, uo, io):
    def tower(packed, sub, par, feats, wf, b, g, beta, out):
        u = lax.bitcast_convert_type(packed[...], jnp.uint32)
        sel = jnp.where(sub[...] > 0, u >> 16, u & jnp.uint32(0xFFFF))
        row = lax.bitcast_convert_type(
            sel.astype(jnp.uint16), jnp.bfloat16).astype(jnp.float32)
        x = (jnp.where(par[...] > 0, row[:, D:], row[:, :D])
             + jnp.dot(feats[...], wf[...], preferred_element_type=jnp.float32)
             + b[...])
        mu = jnp.mean(x, axis=-1, keepdims=True)
        xc = x - mu
        var = jnp.mean(xc * xc, axis=-1, keepdims=True)
        out[...] = xc * lax.rsqrt(var + 1e-5) * g[...] + beta[...]

    tower(up, usub[...], upar[...], uf, wuf, bu, gu, betau, uo)
    tower(ip, isub[...], ipar[...], if_, wif, bi, gi, betai, io)


_BLK = 2048


def _tail(u_pairs, u_sub, u_par, u_feats, i_pairs, i_sub, i_par, i_feats,
          Wuf, bu, Wif, bi, gu, beta_u, gi, beta_i):
    grid = (B // _BLK,)
    pair_spec = pl.BlockSpec((_BLK, 2 * D), lambda i: (i, 0))
    par_spec = pl.BlockSpec((_BLK, 1), lambda i: (i, 0))
    row_spec = pl.BlockSpec((_BLK, D), lambda i: (i, 0))
    ufeat_spec = pl.BlockSpec((_BLK, FU), lambda i: (i, 0))
    ifeat_spec = pl.BlockSpec((_BLK, FI), lambda i: (i, 0))
    full = lambda shape: pl.BlockSpec(shape, lambda i: (0, 0))
    return pl.pallas_call(
        _tail_body,
        grid=grid,
        in_specs=[
            pair_spec, ufeat_spec, par_spec, par_spec,
            pair_spec, ifeat_spec, par_spec, par_spec,
            full((FU, D)), full((1, D)),
            full((FI, D)), full((1, D)),
            full((1, D)), full((1, D)), full((1, D)), full((1, D)),
        ],
        out_specs=[row_spec, row_spec],
        out_shape=[
            jax.ShapeDtypeStruct((B, D), jnp.float32),
            jax.ShapeDtypeStruct((B, D), jnp.float32),
        ],
    )(u_pairs, u_feats, u_sub, u_par, i_pairs, i_feats, i_sub, i_par,
      Wuf, bu, Wif, bi, gu, beta_u, gi, beta_i)


def kernel(user_ids, item_ids, user_feats, item_feats, user_table, item_table,
           Wu, bu, Wi, bi, gu, beta_u, gi, beta_i):
    uid = user_ids.astype(jnp.int32)
    iid = item_ids.astype(jnp.int32)
    pu = _project(user_table.T, Wu[:D])
    pi = _project(item_table.T, Wi[:D])
    u_row = ((uid >> 13) << 11) | (uid & (_PC - 1))
    i_row = ((iid >> 13) << 11) | (iid & (_PC - 1))
    u_pairs = _sc_gather(pu, u_row)
    i_pairs = _sc_gather(pi, i_row)
    u_sub = ((uid >> 12) & 1).reshape(B, 1)
    i_sub = ((iid >> 12) & 1).reshape(B, 1)
    u_par = ((uid >> 11) & 1).reshape(B, 1)
    i_par = ((iid >> 11) & 1).reshape(B, 1)
    return _tail(
        u_pairs, u_sub, u_par, user_feats,
        i_pairs, i_sub, i_par, item_feats,
        Wu[D:], bu.reshape(1, D),
        Wi[D:], bi.reshape(1, D),
        gu.reshape(1, D), beta_u.reshape(1, D),
        gi.reshape(1, D), beta_i.reshape(1, D),
    )
